# baseline (speedup 1.0000x reference)
import functools

import jax
import jax.numpy as jnp
from jax import lax
from jax.experimental import pallas as pl
from jax.experimental.pallas import tpu as pltpu
from jax.experimental.pallas import tpu_sc as plsc

BATCH = 16384
EMBED_DIM = 32
NUM_ROWS = 1000000

_info = plsc.get_sparse_core_info()
_NC, _NS = _info.num_cores, _info.num_subcores
_NW = _NC * _NS
_NWIN = (NUM_ROWS + 127) // 128
_WPW = (_NWIN + _NW - 1) // _NW
_CAP = 768
_OUT_ROWS = BATCH + 2 * _NW

_mesh = plsc.VectorSubcoreMesh(core_axis_name="c", subcore_axis_name="s")


@functools.partial(
    pl.kernel,
    mesh=_mesh,
    out_type=(
        jax.ShapeDtypeStruct((_OUT_ROWS, 128), jnp.float32),
        jax.ShapeDtypeStruct((_OUT_ROWS, 128), jnp.float32),
    ),
    scratch_types=[
        pltpu.VMEM((BATCH,), jnp.int32),
        pltpu.VMEM((_CAP,), jnp.int32),
        pltpu.VMEM((_CAP,), jnp.int32),
        pltpu.VMEM((_CAP, 128), jnp.float32),
        pltpu.VMEM((2, EMBED_DIM, 128), jnp.float32),
        pltpu.SMEM((256,), jnp.int32),
        pltpu.SMEM((_CAP,), jnp.int32),
        pltpu.SemaphoreType.DMA,
        pltpu.SemaphoreType.DMA,
        pltpu.SemaphoreType.DMA,
    ],
    compiler_params=pltpu.CompilerParams(needs_layout_passes=False),
)
def _gather2(
    user_idx_hbm,
    item_idx_hbm,
    user_tab_hbm,
    item_tab_hbm,
    user_out_hbm,
    item_out_hbm,
    idx_v,
    sel_v,
    blist_v,
    rows_v,
    ring_v,
    bins_s,
    sorted_s,
    sem0,
    sem1,
    outsem,
):
    wid = lax.axis_index("s") * _NC + lax.axis_index("c")
    wlo = wid * _WPW
    whi = jnp.minimum(wlo + _WPW, _NWIN)
    dump_row = BATCH + 2 * wid
    iota16 = lax.iota(jnp.int32, 16)
    j_lo = iota16
    j_hi = iota16 + 16
    first_lane = iota16 == 0

    def run_table(idx_hbm, tab_hbm, out_hbm):
        pltpu.sync_copy(idx_hbm, idx_v)
        sentinel = jnp.full((16,), 255 << 7, jnp.int32)
        dump16 = jnp.full((16,), dump_row, jnp.int32)
        for c in range(_CAP // 16):
            sel_v[pl.ds(c * 16, 16)] = sentinel
            blist_v[pl.ds(c * 16, 16)] = dump16

        def sel_body(c, cnt):
            i16 = idx_v[pl.ds(c * 16, 16)]
            win = lax.shift_right_logical(i16, 7)
            m = (win >= wlo) & (win < whi)
            rel = win - wlo
            packed = (
                lax.shift_left((c * 16 + iota16), 15)
                | lax.shift_left(rel, 7)
                | lax.rem(i16, 128)
            )
            cum = plsc.cumsum(jnp.where(m, 1, 0))
            pos = jnp.minimum(cnt + cum - 1, _CAP - 1)
            plsc.store_scatter(sel_v, [pos], packed, mask=m)
            return cnt + cum[15]

        lax.fori_loop(0, BATCH // 16, sel_body, 0)

        def zero_body(r, _):
            bins_s[r] = 0
            return 0

        lax.fori_loop(0, 256, zero_body, 0)

        def count_body(c, _):
            v16 = sel_v[pl.ds(c * 16, 16)]
            for l in range(16):
                rel = lax.rem(lax.shift_right_logical(v16[l], 7), 256)
                bins_s[rel] = bins_s[rel] + 1
            return 0

        lax.fori_loop(0, _CAP // 16, count_body, 0)

        def prefix_body(r, run):
            n = bins_s[r]
            bins_s[r] = run
            return run + n

        lax.fori_loop(0, 256, prefix_body, 0)

        def place_body(c, _):
            v16 = sel_v[pl.ds(c * 16, 16)]
            for l in range(16):
                p = v16[l]
                rel = lax.rem(lax.shift_right_logical(p, 7), 256)
                pos = bins_s[rel]
                bins_s[rel] = pos + 1
                sorted_s[pos] = p
            return 0

        lax.fori_loop(0, _CAP // 16, place_body, 0)

        def fire(w, slot_ref, sem):
            aw = wlo + w
            off = pl.multiple_of(
                jnp.where(aw < _NWIN, lax.shift_left(aw, 7), 0), 128
            )
            return pltpu.async_copy(
                tab_hbm.at[:, pl.ds(off, 128)], slot_ref, sem
            )

        def process(w, slot_ref, start):
            end = bins_s[w]

            def e_body(e, _):
                p = sorted_s[e]
                b16 = jnp.full((16,), lax.shift_right_logical(p, 15), jnp.int32)
                lane16 = jnp.full((16,), lax.rem(p, 128), jnp.int32)
                e16 = jnp.full((16,), e, jnp.int32)
                v_lo = plsc.load_gather(slot_ref, [j_lo, lane16])
                v_hi = plsc.load_gather(slot_ref, [j_hi, lane16])
                plsc.store_scatter(rows_v, [e16, j_lo], v_lo)
                plsc.store_scatter(rows_v, [e16, j_hi], v_hi)
                plsc.store_scatter(blist_v, [e16], b16, mask=first_lane)
                return 0

            lax.fori_loop(start, end, e_body, 0)
            return end

        def wait_slot(slot_ref, sem):
            pltpu.make_async_copy(
                tab_hbm.at[:, pl.ds(pl.multiple_of(0, 128), 128)],
                slot_ref,
                sem,
            ).wait()

        fire(0, ring_v.at[0], sem0)
        fire(1, ring_v.at[1], sem1)

        def pair_body(g, start):
            w_a = 2 * g
            w_b = 2 * g + 1
            wait_slot(ring_v.at[0], sem0)
            start = process(w_a, ring_v.at[0], start)
            fire(w_a + 2, ring_v.at[0], sem0)
            wait_slot(ring_v.at[1], sem1)
            start = process(w_b, ring_v.at[1], start)
            fire(w_b + 2, ring_v.at[1], sem1)
            return start

        start = lax.fori_loop(0, 122, pair_body, 0)
        wait_slot(ring_v.at[0], sem0)
        start = process(244, ring_v.at[0], start)
        wait_slot(ring_v.at[1], sem1)
        start = process(245, ring_v.at[1], start)

        pltpu.async_copy(rows_v, out_hbm.at[blist_v], outsem).wait()

    run_table(user_idx_hbm, user_tab_hbm, user_out_hbm)
    run_table(item_idx_hbm, item_tab_hbm, item_out_hbm)


@jax.jit
def kernel(user_input, item_input, user_table, item_table):
    u, it = _gather2(
        user_input.astype(jnp.int32),
        item_input.astype(jnp.int32),
        user_table.T,
        item_table.T,
    )
    return u[:BATCH, :EMBED_DIM], it[:BATCH, :EMBED_DIM]


# speedup vs baseline: 1.3965x; 1.3965x over previous
import functools

import jax
import jax.numpy as jnp
from jax import lax
from jax.experimental import pallas as pl
from jax.experimental.pallas import tpu as pltpu
from jax.experimental.pallas import tpu_sc as plsc

BATCH = 16384
EMBED_DIM = 32
NUM_ROWS = 1000000

_info = plsc.get_sparse_core_info()
_NC, _NS = _info.num_cores, _info.num_subcores
_NW = _NC * _NS
_NWIN = (NUM_ROWS + 127) // 128
_WPW = (_NWIN + _NW - 1) // _NW
_CAP = 704
_OUT_ROWS = BATCH + 2 * _NW

_mesh = plsc.VectorSubcoreMesh(core_axis_name="c", subcore_axis_name="s")


@functools.partial(
    pl.kernel,
    mesh=_mesh,
    out_type=(
        jax.ShapeDtypeStruct((_OUT_ROWS, 128), jnp.float32),
        jax.ShapeDtypeStruct((_OUT_ROWS, 128), jnp.float32),
    ),
    scratch_types=[
        pltpu.VMEM((2048,), jnp.int32),
        pltpu.VMEM((_CAP,), jnp.int32),
        pltpu.VMEM((_CAP,), jnp.int32),
        pltpu.VMEM((_CAP, 128), jnp.float32),
        pltpu.VMEM((2, EMBED_DIM, 512), jnp.float32),
        pltpu.SMEM((256,), jnp.int32),
        pltpu.SMEM((_CAP,), jnp.int32),
        pltpu.SemaphoreType.DMA,
        pltpu.SemaphoreType.DMA,
        pltpu.SemaphoreType.DMA,
    ],
    compiler_params=pltpu.CompilerParams(needs_layout_passes=False),
)
def _gather2(
    user_idx_hbm,
    item_idx_hbm,
    user_tab_hbm,
    item_tab_hbm,
    user_out_hbm,
    item_out_hbm,
    idx_v,
    sel_v,
    blist_v,
    rows_v,
    ring_v,
    bins_s,
    sorted_s,
    sem0,
    sem1,
    outsem,
):
    wid = lax.axis_index("s") * _NC + lax.axis_index("c")
    wlo = wid * _WPW
    whi = jnp.minimum(wlo + _WPW, _NWIN)
    dump_row = BATCH + 2 * wid
    iota16 = lax.iota(jnp.int32, 16)
    j_lo = iota16
    j_hi = iota16 + 16
    first_lane = iota16 == 0

    def run_table(idx_hbm, tab_hbm, out_hbm):
        sentinel = jnp.full((16,), 255 << 7, jnp.int32)
        dump16 = jnp.full((16,), dump_row, jnp.int32)
        for c in range(_CAP // 16):
            sel_v[pl.ds(c * 16, 16)] = sentinel
            blist_v[pl.ds(c * 16, 16)] = dump16

        def sel_chunk(k, cnt):
            pltpu.sync_copy(idx_hbm.at[pl.ds(k * 2048, 2048)], idx_v)

            def sel_body(c, cnt):
                i16 = idx_v[pl.ds(c * 16, 16)]
                win = lax.shift_right_logical(i16, 7)
                m = (win >= wlo) & (win < whi)
                rel = win - wlo
                packed = (
                    lax.shift_left((k * 2048 + c * 16 + iota16), 15)
                    | lax.shift_left(rel, 7)
                    | lax.rem(i16, 128)
                )
                cum = plsc.cumsum(jnp.where(m, 1, 0))
                pos = jnp.minimum(cnt + cum - 1, _CAP - 1)
                plsc.store_scatter(sel_v, [pos], packed, mask=m)
                return cnt + cum[15]

            return lax.fori_loop(0, 128, sel_body, cnt)

        lax.fori_loop(0, 8, sel_chunk, 0)

        def zero_body(r, _):
            bins_s[r] = 0
            return 0

        lax.fori_loop(0, 256, zero_body, 0)

        def count_body(c, _):
            v16 = sel_v[pl.ds(c * 16, 16)]
            for l in range(16):
                rel = lax.rem(lax.shift_right_logical(v16[l], 7), 256)
                bins_s[rel] = bins_s[rel] + 1
            return 0

        lax.fori_loop(0, _CAP // 16, count_body, 0)

        def prefix_body(r, run):
            n = bins_s[r]
            bins_s[r] = run
            return run + n

        lax.fori_loop(0, 256, prefix_body, 0)

        def place_body(c, _):
            v16 = sel_v[pl.ds(c * 16, 16)]
            for l in range(16):
                p = v16[l]
                rel = lax.rem(lax.shift_right_logical(p, 7), 256)
                pos = bins_s[rel]
                bins_s[rel] = pos + 1
                sorted_s[pos] = p
            return 0

        lax.fori_loop(0, _CAP // 16, place_body, 0)

        _TABW = _NWIN * 128

        def slab_base(s):
            return jnp.minimum(
                lax.shift_left(wlo + 4 * s, 7), _TABW - 512
            )

        def fire(s, slot_ref, sem):
            off = pl.multiple_of(slab_base(s), 128)
            return pltpu.async_copy(
                tab_hbm.at[:, pl.ds(off, 512)], slot_ref, sem
            )

        def process(s, slot_ref, start):
            end = bins_s[4 * s + 3]
            delta = wlo * 128 - slab_base(s)

            def e_body(e, _):
                p = sorted_s[e]
                b16 = jnp.full((16,), lax.shift_right_logical(p, 15), jnp.int32)
                col = lax.rem(p, 32768) + delta
                col16 = jnp.full((16,), col, jnp.int32)
                e16 = jnp.full((16,), e, jnp.int32)
                v_lo = plsc.load_gather(slot_ref, [j_lo, col16])
                v_hi = plsc.load_gather(slot_ref, [j_hi, col16])
                plsc.store_scatter(rows_v, [e16, j_lo], v_lo)
                plsc.store_scatter(rows_v, [e16, j_hi], v_hi)
                plsc.store_scatter(blist_v, [e16], b16, mask=first_lane)
                return 0

            lax.fori_loop(start, end, e_body, 0)
            return end

        def wait_slot(slot_ref, sem):
            pltpu.make_async_copy(
                tab_hbm.at[:, pl.ds(pl.multiple_of(0, 128), 512)],
                slot_ref,
                sem,
            ).wait()

        fire(0, ring_v.at[0], sem0)
        fire(1, ring_v.at[1], sem1)

        def pair_body(g, start):
            s_a = 2 * g
            s_b = 2 * g + 1
            wait_slot(ring_v.at[0], sem0)
            start = process(s_a, ring_v.at[0], start)
            fire(s_a + 2, ring_v.at[0], sem0)
            wait_slot(ring_v.at[1], sem1)
            start = process(s_b, ring_v.at[1], start)
            fire(s_b + 2, ring_v.at[1], sem1)
            return start

        start = lax.fori_loop(0, 30, pair_body, 0)
        wait_slot(ring_v.at[0], sem0)
        start = process(60, ring_v.at[0], start)
        wait_slot(ring_v.at[1], sem1)
        start = process(61, ring_v.at[1], start)

        pltpu.async_copy(rows_v, out_hbm.at[blist_v], outsem).wait()

    run_table(user_idx_hbm, user_tab_hbm, user_out_hbm)
    run_table(item_idx_hbm, item_tab_hbm, item_out_hbm)


@jax.jit
def kernel(user_input, item_input, user_table, item_table):
    u, it = _gather2(
        user_input.astype(jnp.int32),
        item_input.astype(jnp.int32),
        user_table.T,
        item_table.T,
    )
    return u[:BATCH, :EMBED_DIM], it[:BATCH, :EMBED_DIM]


# speedup vs baseline: 1.5179x; 1.0869x over previous
import functools

import jax
import jax.numpy as jnp
from jax import lax
from jax.experimental import pallas as pl
from jax.experimental.pallas import tpu as pltpu
from jax.experimental.pallas import tpu_sc as plsc

BATCH = 16384
EMBED_DIM = 32
NUM_ROWS = 1000000

_info = plsc.get_sparse_core_info()
_NC, _NS = _info.num_cores, _info.num_subcores
_NW = _NC * _NS
_NWIN = (NUM_ROWS + 127) // 128
_WPW = (_NWIN + _NW - 1) // _NW
_CAP = 768
_OUT_ROWS = BATCH + 2 * _NW

_mesh = plsc.VectorSubcoreMesh(core_axis_name="c", subcore_axis_name="s")


@functools.partial(
    pl.kernel,
    mesh=_mesh,
    out_type=(
        jax.ShapeDtypeStruct((_OUT_ROWS, 128), jnp.float32),
        jax.ShapeDtypeStruct((_OUT_ROWS, 128), jnp.float32),
    ),
    scratch_types=[
        pltpu.VMEM((2048,), jnp.int32),
        pltpu.VMEM((_CAP,), jnp.int32),
        pltpu.VMEM((_CAP // 128, 128), jnp.int32),
        pltpu.VMEM((2, 128, 128), jnp.float32),
        pltpu.VMEM((4, EMBED_DIM, 512), jnp.float32),
        pltpu.SMEM((256,), jnp.int32),
        pltpu.SMEM((_CAP,), jnp.int32),
        pltpu.SemaphoreType.DMA,
        pltpu.SemaphoreType.DMA,
        pltpu.SemaphoreType.DMA,
        pltpu.SemaphoreType.DMA,
        pltpu.SemaphoreType.DMA,
    ],
    compiler_params=pltpu.CompilerParams(needs_layout_passes=False),
)
def _gather2(
    user_idx_hbm,
    item_idx_hbm,
    user_tab_hbm,
    item_tab_hbm,
    user_out_hbm,
    item_out_hbm,
    idx_v,
    sel_v,
    blist_v,
    rows_v,
    ring_v,
    bins_s,
    sorted_s,
    sem0,
    sem1,
    sem2,
    sem3,
    outsem,
):
    wid = lax.axis_index("s") * _NC + lax.axis_index("c")
    wlo = wid * _WPW
    whi = jnp.minimum(wlo + _WPW, _NWIN)
    dump_row = BATCH + 2 * wid
    iota16 = lax.iota(jnp.int32, 16)
    j_lo = iota16
    j_hi = iota16 + 16
    first_lane = iota16 == 0

    def run_table(idx_hbm, tab_hbm, out_hbm):
        sentinel = jnp.full((16,), 255 << 7, jnp.int32)
        dump16 = jnp.full((16,), dump_row, jnp.int32)
        for c in range(_CAP // 16):
            sel_v[pl.ds(c * 16, 16)] = sentinel
        for c in range(_CAP // 128):
            c16 = jnp.full((16,), c, jnp.int32)
            for q in range(8):
                plsc.store_scatter(blist_v, [c16, q * 16 + iota16], dump16)

        def sel_chunk(k, cnt):
            pltpu.sync_copy(idx_hbm.at[pl.ds(k * 2048, 2048)], idx_v)

            def sel_body(c, cnt):
                i16 = idx_v[pl.ds(c * 16, 16)]
                win = lax.shift_right_logical(i16, 7)
                m = (win >= wlo) & (win < whi)
                rel = win - wlo
                packed = (
                    lax.shift_left((k * 2048 + c * 16 + iota16), 15)
                    | lax.shift_left(rel, 7)
                    | lax.rem(i16, 128)
                )
                cum = plsc.cumsum(jnp.where(m, 1, 0))
                pos = jnp.minimum(cnt + cum - 1, _CAP - 1)
                plsc.store_scatter(sel_v, [pos], packed, mask=m)
                return cnt + cum[15]

            return lax.fori_loop(0, 128, sel_body, cnt)

        lax.fori_loop(0, 8, sel_chunk, 0)

        def zero_body(r, _):
            bins_s[r] = 0
            return 0

        lax.fori_loop(0, 256, zero_body, 0)

        def count_body(c, _):
            v16 = sel_v[pl.ds(c * 16, 16)]
            for l in range(16):
                rel = lax.rem(lax.shift_right_logical(v16[l], 7), 256)
                bins_s[rel] = bins_s[rel] + 1
            return 0

        lax.fori_loop(0, _CAP // 16, count_body, 0)

        def prefix_body(r, run):
            n = bins_s[r]
            bins_s[r] = run
            return run + n

        lax.fori_loop(0, 256, prefix_body, 0)

        def place_body(c, _):
            v16 = sel_v[pl.ds(c * 16, 16)]
            for l in range(16):
                p = v16[l]
                rel = lax.rem(lax.shift_right_logical(p, 7), 256)
                pos = bins_s[rel]
                bins_s[rel] = pos + 1
                sorted_s[pos] = p
            return 0

        lax.fori_loop(0, _CAP // 16, place_body, 0)

        _TABW = _NWIN * 128
        sems = (sem0, sem1, sem2, sem3)

        def slab_base(s):
            return jnp.minimum(
                lax.shift_left(wlo + 4 * s, 7), _TABW - 512
            )

        def fire(s, slot, sem):
            off = pl.multiple_of(slab_base(s), 128)
            return pltpu.async_copy(
                tab_hbm.at[:, pl.ds(off, 512)], ring_v.at[slot], sem
            )

        def flush_chunk(c):
            @pl.when(lax.rem(c, 2) == 0)
            def _():
                pltpu.async_copy(
                    rows_v.at[0], out_hbm.at[blist_v.at[c]], outsem
                ).wait()

            @pl.when(lax.rem(c, 2) == 1)
            def _():
                pltpu.async_copy(
                    rows_v.at[1], out_hbm.at[blist_v.at[c]], outsem
                ).wait()

        def process(s, slot, carry):
            start, fc = carry
            end = bins_s[jnp.minimum(4 * s + 3, 254)]
            delta = wlo * 128 - slab_base(s)
            slot_ref = ring_v.at[slot]

            def e_body(e, _):
                p = sorted_s[e]
                b16 = jnp.full((16,), lax.shift_right_logical(p, 15), jnp.int32)
                col = lax.rem(p, 32768) + delta
                col16 = jnp.full((16,), col, jnp.int32)
                par16 = jnp.full((16,), lax.rem(lax.div(e, 128), 2), jnp.int32)
                chunk16 = jnp.full((16,), lax.div(e, 128), jnp.int32)
                slot16 = jnp.full((16,), lax.rem(e, 128), jnp.int32)
                v_lo = plsc.load_gather(slot_ref, [j_lo, col16])
                v_hi = plsc.load_gather(slot_ref, [j_hi, col16])
                plsc.store_scatter(rows_v, [par16, slot16, j_lo], v_lo)
                plsc.store_scatter(rows_v, [par16, slot16, j_hi], v_hi)
                plsc.store_scatter(
                    blist_v, [chunk16, slot16], b16, mask=first_lane
                )
                return 0

            lax.fori_loop(start, end, e_body, 0)

            def flush_body(c, _):
                flush_chunk(c)
                return 0

            nc = lax.div(end, 128)
            lax.fori_loop(fc, nc, flush_body, 0)
            return end, nc

        def wait_slot(slot, sem):
            pltpu.make_async_copy(
                tab_hbm.at[:, pl.ds(pl.multiple_of(0, 128), 512)],
                ring_v.at[slot],
                sem,
            ).wait()

        for k in range(4):
            fire(k, k, sems[k])

        def quad_body(g, carry):
            for k in range(4):
                wait_slot(k, sems[k])
                carry = process(4 * g + k, k, carry)
                fire(4 * g + 4 + k, k, sems[k])
            return carry

        carry = lax.fori_loop(0, 15, quad_body, (0, 0))
        for k in range(4):
            wait_slot(k, sems[k])
            carry = process(60 + k, k, carry)

        _, fc = carry

        def tail_flush(c, _):
            flush_chunk(c)
            return 0

        lax.fori_loop(fc, _CAP // 128, tail_flush, 0)

    run_table(user_idx_hbm, user_tab_hbm, user_out_hbm)
    run_table(item_idx_hbm, item_tab_hbm, item_out_hbm)


@jax.jit
def kernel(user_input, item_input, user_table, item_table):
    u, it = _gather2(
        user_input.astype(jnp.int32),
        item_input.astype(jnp.int32),
        user_table.T,
        item_table.T,
    )
    return u[:BATCH, :EMBED_DIM], it[:BATCH, :EMBED_DIM]


# speedup vs baseline: 1.5234x; 1.0036x over previous
import functools

import jax
import jax.numpy as jnp
from jax import lax
from jax.experimental import pallas as pl
from jax.experimental.pallas import tpu as pltpu
from jax.experimental.pallas import tpu_sc as plsc

BATCH = 16384
EMBED_DIM = 32
NUM_ROWS = 1000000

_info = plsc.get_sparse_core_info()
_NC, _NS = _info.num_cores, _info.num_subcores
_NW = _NC * _NS
_NWIN = (NUM_ROWS + 127) // 128
_WPW = (_NWIN + _NW - 1) // _NW
_CAP = 768
_OUT_ROWS = BATCH + 2 * _NW

_mesh = plsc.VectorSubcoreMesh(core_axis_name="c", subcore_axis_name="s")


@functools.partial(
    pl.kernel,
    mesh=_mesh,
    out_type=(
        jax.ShapeDtypeStruct((_OUT_ROWS, 128), jnp.float32),
        jax.ShapeDtypeStruct((_OUT_ROWS, 128), jnp.float32),
    ),
    scratch_types=[
        pltpu.VMEM((2048,), jnp.int32),
        pltpu.VMEM((_CAP,), jnp.int32),
        pltpu.VMEM((_CAP // 128, 128), jnp.int32),
        pltpu.VMEM((2, 128, 128), jnp.float32),
        pltpu.VMEM((4, EMBED_DIM, 512), jnp.float32),
        pltpu.SMEM((256,), jnp.int32),
        pltpu.SMEM((_CAP,), jnp.int32),
        pltpu.SemaphoreType.DMA,
        pltpu.SemaphoreType.DMA,
        pltpu.SemaphoreType.DMA,
        pltpu.SemaphoreType.DMA,
        pltpu.SemaphoreType.DMA,
    ],
    compiler_params=pltpu.CompilerParams(needs_layout_passes=False),
)
def _gather2(
    user_idx_hbm,
    item_idx_hbm,
    user_tab_hbm,
    item_tab_hbm,
    user_out_hbm,
    item_out_hbm,
    idx_v,
    sel_v,
    blist_v,
    rows_v,
    ring_v,
    bins_s,
    sorted_s,
    sem0,
    sem1,
    sem2,
    sem3,
    outsem,
):
    wid = lax.axis_index("s") * _NC + lax.axis_index("c")
    wlo = wid * _WPW
    whi = jnp.minimum(wlo + _WPW, _NWIN)
    dump_row = BATCH + 2 * wid
    iota16 = lax.iota(jnp.int32, 16)
    j_lo = iota16
    j_hi = iota16 + 16
    first_lane = iota16 == 0

    def run_table(idx_hbm, tab_hbm, out_hbm):
        sentinel = jnp.full((16,), 255 << 7, jnp.int32)
        dump16 = jnp.full((16,), dump_row, jnp.int32)
        for c in range(_CAP // 16):
            sel_v[pl.ds(c * 16, 16)] = sentinel
        for c in range(_CAP // 128):
            c16 = jnp.full((16,), c, jnp.int32)
            for q in range(8):
                plsc.store_scatter(blist_v, [c16, q * 16 + iota16], dump16)

        def sel_chunk(k, cnt):
            pltpu.sync_copy(idx_hbm.at[pl.ds(k * 2048, 2048)], idx_v)

            def sel_body(c, cnt):
                i16 = idx_v[pl.ds(c * 16, 16)]
                win = lax.shift_right_logical(i16, 7)
                m = (win >= wlo) & (win < whi)
                rel = win - wlo
                packed = (
                    lax.shift_left((k * 2048 + c * 16 + iota16), 15)
                    | lax.shift_left(rel, 7)
                    | lax.rem(i16, 128)
                )
                cum = plsc.cumsum(jnp.where(m, 1, 0))
                pos = jnp.minimum(cnt + cum - 1, _CAP - 1)
                plsc.store_scatter(sel_v, [pos], packed, mask=m)
                return cnt + cum[15]

            return lax.fori_loop(0, 128, sel_body, cnt)

        lax.fori_loop(0, 8, sel_chunk, 0)

        def zero_body(r, _):
            bins_s[r] = 0
            return 0

        lax.fori_loop(0, 256, zero_body, 0)

        def count_body(c, _):
            v16 = sel_v[pl.ds(c * 16, 16)]
            for l in range(16):
                rel = lax.rem(lax.shift_right_logical(v16[l], 7), 256)
                bins_s[rel] = bins_s[rel] + 1
            return 0

        lax.fori_loop(0, _CAP // 16, count_body, 0)

        def prefix_body(r, run):
            n = bins_s[r]
            bins_s[r] = run
            return run + n

        lax.fori_loop(0, 256, prefix_body, 0)

        def place_body(c, _):
            v16 = sel_v[pl.ds(c * 16, 16)]
            for l in range(16):
                p = v16[l]
                rel = lax.rem(lax.shift_right_logical(p, 7), 256)
                pos = bins_s[rel]
                bins_s[rel] = pos + 1
                sorted_s[pos] = p
            return 0

        lax.fori_loop(0, _CAP // 16, place_body, 0)

        _TABW = _NWIN * 128
        sems = (sem0, sem1, sem2, sem3)

        def slab_base(s):
            return jnp.minimum(
                lax.shift_left(wlo + 4 * s, 7), _TABW - 512
            )

        def fire(s, slot, sem):
            off = pl.multiple_of(slab_base(s), 128)
            return pltpu.async_copy(
                tab_hbm.at[:, pl.ds(off, 512)], ring_v.at[slot], sem
            )

        def drain_flush():
            pltpu.make_async_copy(
                rows_v.at[0], out_hbm.at[blist_v.at[0]], outsem
            ).wait()

        def flush_chunk(c):
            @pl.when(c > 0)
            def _():
                drain_flush()

            @pl.when(lax.rem(c, 2) == 0)
            def _():
                pltpu.async_copy(
                    rows_v.at[0], out_hbm.at[blist_v.at[c]], outsem
                )

            @pl.when(lax.rem(c, 2) == 1)
            def _():
                pltpu.async_copy(
                    rows_v.at[1], out_hbm.at[blist_v.at[c]], outsem
                )

        def process(s, slot, carry):
            start, fc = carry
            end = bins_s[jnp.minimum(4 * s + 3, 254)]
            delta = wlo * 128 - slab_base(s)
            slot_ref = ring_v.at[slot]

            def e_body(e, _):
                p = sorted_s[e]
                b16 = jnp.full((16,), lax.shift_right_logical(p, 15), jnp.int32)
                col = lax.rem(p, 32768) + delta
                col16 = jnp.full((16,), col, jnp.int32)
                par16 = jnp.full((16,), lax.rem(lax.div(e, 128), 2), jnp.int32)
                chunk16 = jnp.full((16,), lax.div(e, 128), jnp.int32)
                slot16 = jnp.full((16,), lax.rem(e, 128), jnp.int32)
                v_lo = plsc.load_gather(slot_ref, [j_lo, col16])
                v_hi = plsc.load_gather(slot_ref, [j_hi, col16])
                plsc.store_scatter(rows_v, [par16, slot16, j_lo], v_lo)
                plsc.store_scatter(rows_v, [par16, slot16, j_hi], v_hi)
                plsc.store_scatter(
                    blist_v, [chunk16, slot16], b16, mask=first_lane
                )
                return 0

            lax.fori_loop(start, end, e_body, 0)

            def flush_body(c, _):
                flush_chunk(c)
                return 0

            nc = lax.div(end, 128)
            lax.fori_loop(fc, nc, flush_body, 0)
            return end, nc

        def wait_slot(slot, sem):
            pltpu.make_async_copy(
                tab_hbm.at[:, pl.ds(pl.multiple_of(0, 128), 512)],
                ring_v.at[slot],
                sem,
            ).wait()

        for k in range(4):
            fire(k, k, sems[k])

        def quad_body(g, carry):
            for k in range(4):
                wait_slot(k, sems[k])
                carry = process(4 * g + k, k, carry)
                fire(4 * g + 4 + k, k, sems[k])
            return carry

        carry = lax.fori_loop(0, 15, quad_body, (0, 0))
        for k in range(4):
            wait_slot(k, sems[k])
            carry = process(60 + k, k, carry)

        _, fc = carry

        def tail_flush(c, _):
            flush_chunk(c)
            return 0

        lax.fori_loop(fc, _CAP // 128, tail_flush, 0)
        drain_flush()

    run_table(user_idx_hbm, user_tab_hbm, user_out_hbm)
    run_table(item_idx_hbm, item_tab_hbm, item_out_hbm)


@jax.jit
def kernel(user_input, item_input, user_table, item_table):
    u, it = _gather2(
        user_input.astype(jnp.int32),
        item_input.astype(jnp.int32),
        user_table.T,
        item_table.T,
    )
    return u[:BATCH, :EMBED_DIM], it[:BATCH, :EMBED_DIM]
